# baseline (speedup 1.0000x reference)
import jax
import jax.numpy as jnp
from jax.experimental import pallas as pl

_BLOCK = 2048


def _body(zu_ref, zi_ref, p_ref, a_ref, rel_ref, pui_ref, xui_ref):
    zu_b = zu_ref[...]
    zi_b = zi_ref[...]
    p = p_ref[...]
    a = a_ref[...]
    rel = rel_ref[...]
    d = zu_b.shape[1]
    u0 = jnp.dot(zu_b, p[:d, :], preferred_element_type=jnp.float32)
    u1 = jnp.dot(zu_b, p[d:, :], preferred_element_type=jnp.float32)
    t0 = jnp.sum(u0 * zi_b, axis=1, keepdims=True)
    t1 = jnp.sum(u1 * zi_b, axis=1, keepdims=True)
    pui = t0 * a[0:1, :] + t1 * a[1:2, :]
    m = jnp.max(pui, axis=1, keepdims=True)
    e = jnp.exp(pui - m)
    s = jnp.sum(e, axis=1, keepdims=True)
    x = jnp.sum(e * rel, axis=1, keepdims=True) / s
    pui_ref[...] = pui
    xui_ref[...] = x


def kernel(zu, zi, P, A, relations):
    b, d = zu.shape
    r = A.shape[0]
    nb = A.shape[1]
    p2 = P.reshape(nb * d, d)
    a2 = A[:, :, 0].T
    rel2 = relations.reshape(1, r)
    grid = b // _BLOCK
    pui, xui = pl.pallas_call(
        _body,
        grid=(grid,),
        in_specs=[
            pl.BlockSpec((_BLOCK, d), lambda i: (i, 0)),
            pl.BlockSpec((_BLOCK, d), lambda i: (i, 0)),
            pl.BlockSpec((nb * d, d), lambda i: (0, 0)),
            pl.BlockSpec((nb, r), lambda i: (0, 0)),
            pl.BlockSpec((1, r), lambda i: (0, 0)),
        ],
        out_specs=[
            pl.BlockSpec((_BLOCK, r), lambda i: (i, 0)),
            pl.BlockSpec((_BLOCK, 1), lambda i: (i, 0)),
        ],
        out_shape=[
            jax.ShapeDtypeStruct((b, r), jnp.float32),
            jax.ShapeDtypeStruct((b, 1), jnp.float32),
        ],
    )(zu, zi, p2, a2, rel2)
    return (xui[:, 0], pui)


# speedup vs baseline: 1.0234x; 1.0234x over previous
import jax
import jax.numpy as jnp
from jax.experimental import pallas as pl

_BLOCK = 8192


def _body(zu_ref, zi_ref, p_ref, a_ref, rel_ref, pui_ref, xui_ref):
    zu_b = zu_ref[...]
    zi_b = zi_ref[...]
    p = p_ref[...]
    a = a_ref[...]
    rel = rel_ref[...]
    d = zu_b.shape[1]
    u0 = jnp.dot(zu_b, p[:d, :], preferred_element_type=jnp.float32)
    u1 = jnp.dot(zu_b, p[d:, :], preferred_element_type=jnp.float32)
    t0 = jnp.sum(u0 * zi_b, axis=1, keepdims=True)
    t1 = jnp.sum(u1 * zi_b, axis=1, keepdims=True)
    pui = t0 * a[0:1, :] + t1 * a[1:2, :]
    m = jnp.max(pui, axis=1, keepdims=True)
    e = jnp.exp(pui - m)
    s = jnp.sum(e, axis=1, keepdims=True)
    x = jnp.sum(e * rel, axis=1, keepdims=True) / s
    pui_ref[...] = pui
    xui_ref[...] = x


def kernel(zu, zi, P, A, relations):
    b, d = zu.shape
    r = A.shape[0]
    nb = A.shape[1]
    p2 = P.reshape(nb * d, d)
    a2 = A[:, :, 0].T
    rel2 = relations.reshape(1, r)
    grid = b // _BLOCK
    pui, xui = pl.pallas_call(
        _body,
        grid=(grid,),
        in_specs=[
            pl.BlockSpec((_BLOCK, d), lambda i: (i, 0)),
            pl.BlockSpec((_BLOCK, d), lambda i: (i, 0)),
            pl.BlockSpec((nb * d, d), lambda i: (0, 0)),
            pl.BlockSpec((nb, r), lambda i: (0, 0)),
            pl.BlockSpec((1, r), lambda i: (0, 0)),
        ],
        out_specs=[
            pl.BlockSpec((_BLOCK, r), lambda i: (i, 0)),
            pl.BlockSpec((_BLOCK, 1), lambda i: (i, 0)),
        ],
        out_shape=[
            jax.ShapeDtypeStruct((b, r), jnp.float32),
            jax.ShapeDtypeStruct((b, 1), jnp.float32),
        ],
    )(zu, zi, p2, a2, rel2)
    return (xui[:, 0], pui)
